# baseline (speedup 1.0000x reference)
import functools

import jax
import jax.numpy as jnp
from jax.experimental import pallas as pl
from jax.experimental.pallas import tpu as pltpu
from jax.experimental.pallas import tpu_sc as plsc

N = 784
NPAD = 896
C = 96
CP = 128
E = 64
HID = 192
CAP = 24
RB = 8
PRIM = E * CAP
OVFSLOTS = 1208
NCHUNK_OVF = OVFSLOTS // RB
OVF_BASE = PRIM
TRASH = PRIM + OVFSLOTS
NBUF = TRASH + (NPAD - N)


def _meta_body(tok_ref, wr_ref, br_ref, dest_ref, co_ref):
    logits = jax.lax.dot_general(
        wr_ref[:], tok_ref[:, :C], (((1,), (1,)), ((), ())),
        preferred_element_type=jnp.float32) + br_ref[:]
    maxv = jnp.max(logits, axis=0, keepdims=True)
    sub = jax.lax.broadcasted_iota(jnp.int32, (E, NPAD), 0)
    eidx = jnp.min(jnp.where(logits >= maxv, sub, E), axis=0,
                   keepdims=True)
    lane = jax.lax.broadcasted_iota(jnp.int32, (E, NPAD), 1)
    onehot = ((sub == eidx) & (lane < N)).astype(jnp.float32)

    rp = jax.lax.broadcasted_iota(jnp.int32, (NPAD, NPAD), 0)
    rq = jax.lax.broadcasted_iota(jnp.int32, (NPAD, NPAD), 1)
    ut = (rp < rq).astype(jnp.float32)
    cum = jax.lax.dot_general(
        onehot, ut, (((1,), (0,)), ((), ())),
        preferred_element_type=jnp.float32)
    rank_row = jnp.sum(onehot * cum, axis=0, keepdims=True)

    counts = jnp.sum(onehot, axis=1, keepdims=True)
    ovf = jnp.maximum(counts - float(CAP), 0.0)
    pco = jnp.floor((ovf + 7.0) * 0.125) * 8.0
    r64 = jax.lax.broadcasted_iota(jnp.int32, (E, E), 0)
    c64 = jax.lax.broadcasted_iota(jnp.int32, (E, E), 1)
    lt = (c64 < r64).astype(jnp.float32)
    offo = jax.lax.dot_general(
        lt, pco, (((1,), (0,)), ((), ())),
        preferred_element_type=jnp.float32)
    offo_row = jnp.sum(onehot * offo, axis=0, keepdims=True)
    eidx_f = jnp.sum(onehot * sub.astype(jnp.float32), axis=0,
                     keepdims=True)

    prim = eidx_f * float(CAP) + rank_row
    ovfd = float(OVF_BASE) + offo_row + rank_row - float(CAP)
    dest = jnp.where(rank_row < float(CAP), prim, ovfd).astype(jnp.int32)
    lane1 = jax.lax.broadcasted_iota(jnp.int32, (1, NPAD), 1)
    dest_ref[:] = jnp.where(lane1 < N, dest, TRASH + lane1 - N)

    offo_end = (offo + pco).astype(jnp.int32)
    cj = jax.lax.broadcasted_iota(jnp.int32, (E, NCHUNK_OVF), 1) * RB
    ce = jnp.sum((offo_end <= cj).astype(jnp.int32), axis=0, keepdims=True)
    ce = jnp.minimum(ce, E - 1)
    novf = jnp.sum(pco, axis=0, keepdims=True).astype(jnp.int32) // RB
    lanec = jax.lax.broadcasted_iota(jnp.int32, (1, NCHUNK_OVF + 1), 1)
    co_ref[:] = jnp.where(lanec < NCHUNK_OVF,
                          jnp.pad(ce, ((0, 0), (0, 1))),
                          jnp.broadcast_to(novf, (1, NCHUNK_OVF + 1)))


def _router_meta(tokens_pad, Wr, br):
    return pl.pallas_call(
        _meta_body,
        in_specs=[
            pl.BlockSpec((NPAD, CP), lambda: (0, 0)),
            pl.BlockSpec((E, C), lambda: (0, 0)),
            pl.BlockSpec((E, 1), lambda: (0, 0)),
        ],
        out_specs=[
            pl.BlockSpec((1, NPAD), lambda: (0, 0)),
            pl.BlockSpec((1, NCHUNK_OVF + 1), lambda: (0, 0)),
        ],
        out_shape=[
            jax.ShapeDtypeStruct((1, NPAD), jnp.int32),
            jax.ShapeDtypeStruct((1, NCHUNK_OVF + 1), jnp.int32),
        ],
    )(tokens_pad, Wr, br.reshape(E, 1))


def _sc_scatter(tokens_pad, dest):
    mesh = plsc.VectorSubcoreMesh(core_axis_name="c", subcore_axis_name="s")

    @functools.partial(
        pl.kernel,
        out_type=jax.ShapeDtypeStruct((NBUF, CP), jnp.float32),
        mesh=mesh)
    def k(x_hbm, i_hbm, o_hbm):
        def body(x_vmem, i_vmem):
            pltpu.sync_copy(x_vmem, o_hbm.at[i_vmem.at[0]])

        pltpu.emit_pipeline(
            body,
            grid=(NPAD // 128,),
            in_specs=[
                pl.BlockSpec((128, CP), lambda i: (i, 0)),
                pl.BlockSpec((1, 128), lambda i: (0, i)),
            ],
            out_specs=[],
            core_axis_name=("c", "s"),
            dimension_semantics=(pltpu.PARALLEL,),
        )(x_hbm, i_hbm)

    return k(tokens_pad, dest)


def _sc_gather(y_sorted, dest):
    mesh = plsc.VectorSubcoreMesh(core_axis_name="c", subcore_axis_name="s")

    @functools.partial(
        pl.kernel,
        out_type=jax.ShapeDtypeStruct((NPAD, CP), jnp.float32),
        mesh=mesh)
    def k(y_hbm, i_hbm, o_hbm):
        def body(i_vmem, o_vmem):
            pltpu.sync_copy(y_hbm.at[i_vmem.at[0]], o_vmem)

        pltpu.emit_pipeline(
            body,
            grid=(NPAD // 128,),
            in_specs=[pl.BlockSpec((1, 128), lambda i: (0, i))],
            out_specs=[pl.BlockSpec((128, CP), lambda i: (i, 0))],
            core_axis_name=("c", "s"),
            dimension_semantics=(pltpu.PARALLEL,),
        )(i_hbm, o_hbm)

    return k(y_sorted, dest)


def _expert_body(co_ref, tok_ref, w1_ref, b1_ref, w2_ref, b2_ref,
                 scale_ref, out_ref):
    scale = scale_ref[0, 0]

    def mlp(t, e_w1, e_b1, e_w2, e_b2):
        h1 = jax.lax.dot_general(
            t.astype(jnp.bfloat16), e_w1, (((1,), (0,)), ((), ())),
            preferred_element_type=jnp.float32) + e_b1
        h1 = 0.5 * h1 * (1.0 + jax.lax.erf(h1 * 0.7071067811865476))
        ye = jax.lax.dot_general(
            h1.astype(jnp.bfloat16), e_w2, (((1,), (0,)), ((), ())),
            preferred_element_type=jnp.float32) + e_b2
        return t + scale * ye

    for e in range(E):
        t = tok_ref[pl.ds(e * CAP, CAP), :C]
        out_ref[pl.ds(e * CAP, CAP), :C] = mlp(
            t, w1_ref[e], b1_ref[e], w2_ref[e], b2_ref[e])

    novf = co_ref[NCHUNK_OVF]

    def step(j, carry):
        e = co_ref[j]
        t = tok_ref[pl.ds(OVF_BASE + j * RB, RB), :C]
        out_ref[pl.ds(OVF_BASE + j * RB, RB), :C] = mlp(
            t, w1_ref[e], b1_ref[e], w2_ref[e], b2_ref[e])
        return carry

    jax.lax.fori_loop(0, novf, step, 0)


def _expert_compute(co, sorted_tokens, W1, b1, W2, b2, scale):
    grid_spec = pltpu.PrefetchScalarGridSpec(
        num_scalar_prefetch=1,
        grid=(1,),
        in_specs=[
            pl.BlockSpec((NBUF, CP), lambda i, co: (0, 0)),
            pl.BlockSpec((E, C, HID), lambda i, co: (0, 0, 0)),
            pl.BlockSpec((E, 1, HID), lambda i, co: (0, 0, 0)),
            pl.BlockSpec((E, HID, C), lambda i, co: (0, 0, 0)),
            pl.BlockSpec((E, 1, C), lambda i, co: (0, 0, 0)),
            pl.BlockSpec((1, 1), lambda i, co: (0, 0)),
        ],
        out_specs=pl.BlockSpec((NBUF, CP), lambda i, co: (0, 0)),
    )
    return pl.pallas_call(
        _expert_body,
        grid_spec=grid_spec,
        out_shape=jax.ShapeDtypeStruct((NBUF, CP), jnp.float32),
    )(co, sorted_tokens,
      jnp.transpose(W1.astype(jnp.bfloat16), (0, 2, 1)),
      b1.reshape(E, 1, HID),
      jnp.transpose(W2.astype(jnp.bfloat16), (0, 2, 1)),
      b2.reshape(E, 1, C), scale.reshape(1, 1))


def kernel(x, Wr, br, W1, b1, W2, b2, scale):
    b, c, h, w = x.shape
    tokens = jnp.transpose(x, (0, 2, 3, 1)).reshape(b * h * w, c)
    tokens_pad = jnp.zeros((NPAD, CP), jnp.float32).at[:N, :C].set(tokens)

    dest, co = _router_meta(tokens_pad, Wr, br)
    sorted_tokens = _sc_scatter(tokens_pad, dest)
    y_sorted = _expert_compute(co.reshape(NCHUNK_OVF + 1), sorted_tokens,
                               W1, b1, W2, b2, scale)
    y_tokens = _sc_gather(y_sorted, dest)

    return jnp.transpose(y_tokens[:N, :C].reshape(b, h, w, c), (0, 3, 1, 2))


# speedup vs baseline: 1.0116x; 1.0116x over previous
import functools

import jax
import jax.numpy as jnp
from jax.experimental import pallas as pl
from jax.experimental.pallas import tpu as pltpu
from jax.experimental.pallas import tpu_sc as plsc

N = 784
NPAD = 896
C = 96
CP = 128
E = 64
HID = 192
CAP = 32
RB = 8
PRIM = E * CAP
OVFSLOTS = 1216
NCHUNK_OVF = OVFSLOTS // RB
OVF_BASE = PRIM
TRASH = PRIM + OVFSLOTS
NBUF = TRASH + (NPAD - N)


def _meta_body(tok_ref, wr_ref, br_ref, dest_ref, co_ref):
    logits = jax.lax.dot_general(
        wr_ref[:], tok_ref[:, :C], (((1,), (1,)), ((), ())),
        preferred_element_type=jnp.float32) + br_ref[:]
    maxv = jnp.max(logits, axis=0, keepdims=True)
    sub = jax.lax.broadcasted_iota(jnp.int32, (E, NPAD), 0)
    eidx = jnp.min(jnp.where(logits >= maxv, sub, E), axis=0,
                   keepdims=True)
    lane = jax.lax.broadcasted_iota(jnp.int32, (E, NPAD), 1)
    onehot = ((sub == eidx) & (lane < N)).astype(jnp.float32)

    rp = jax.lax.broadcasted_iota(jnp.int32, (NPAD, NPAD), 0)
    rq = jax.lax.broadcasted_iota(jnp.int32, (NPAD, NPAD), 1)
    ut = (rp < rq).astype(jnp.float32)
    cum = jax.lax.dot_general(
        onehot, ut, (((1,), (0,)), ((), ())),
        preferred_element_type=jnp.float32)
    rank_row = jnp.sum(onehot * cum, axis=0, keepdims=True)

    counts = jnp.sum(onehot, axis=1, keepdims=True)
    ovf = jnp.maximum(counts - float(CAP), 0.0)
    pco = jnp.floor((ovf + 7.0) * 0.125) * 8.0
    r64 = jax.lax.broadcasted_iota(jnp.int32, (E, E), 0)
    c64 = jax.lax.broadcasted_iota(jnp.int32, (E, E), 1)
    lt = (c64 < r64).astype(jnp.float32)
    offo = jax.lax.dot_general(
        lt, pco, (((1,), (0,)), ((), ())),
        preferred_element_type=jnp.float32)
    offo_row = jnp.sum(onehot * offo, axis=0, keepdims=True)
    eidx_f = jnp.sum(onehot * sub.astype(jnp.float32), axis=0,
                     keepdims=True)

    prim = eidx_f * float(CAP) + rank_row
    ovfd = float(OVF_BASE) + offo_row + rank_row - float(CAP)
    dest = jnp.where(rank_row < float(CAP), prim, ovfd).astype(jnp.int32)
    lane1 = jax.lax.broadcasted_iota(jnp.int32, (1, NPAD), 1)
    dest_ref[:] = jnp.where(lane1 < N, dest, TRASH + lane1 - N)

    offo_end = (offo + pco).astype(jnp.int32)
    cj = jax.lax.broadcasted_iota(jnp.int32, (E, NCHUNK_OVF), 1) * RB
    ce = jnp.sum((offo_end <= cj).astype(jnp.int32), axis=0, keepdims=True)
    ce = jnp.minimum(ce, E - 1)
    novf = jnp.sum(pco, axis=0, keepdims=True).astype(jnp.int32) // RB
    lanec = jax.lax.broadcasted_iota(jnp.int32, (1, NCHUNK_OVF + 1), 1)
    co_ref[:] = jnp.where(lanec < NCHUNK_OVF,
                          jnp.pad(ce, ((0, 0), (0, 1))),
                          jnp.broadcast_to(novf, (1, NCHUNK_OVF + 1)))


def _router_meta(tokens_pad, Wr, br):
    return pl.pallas_call(
        _meta_body,
        in_specs=[
            pl.BlockSpec((NPAD, CP), lambda: (0, 0)),
            pl.BlockSpec((E, C), lambda: (0, 0)),
            pl.BlockSpec((E, 1), lambda: (0, 0)),
        ],
        out_specs=[
            pl.BlockSpec((1, NPAD), lambda: (0, 0)),
            pl.BlockSpec((1, NCHUNK_OVF + 1), lambda: (0, 0)),
        ],
        out_shape=[
            jax.ShapeDtypeStruct((1, NPAD), jnp.int32),
            jax.ShapeDtypeStruct((1, NCHUNK_OVF + 1), jnp.int32),
        ],
    )(tokens_pad, Wr, br.reshape(E, 1))


def _sc_scatter(tokens_pad, dest):
    mesh = plsc.VectorSubcoreMesh(core_axis_name="c", subcore_axis_name="s")

    @functools.partial(
        pl.kernel,
        out_type=jax.ShapeDtypeStruct((NBUF, CP), jnp.float32),
        mesh=mesh)
    def k(x_hbm, i_hbm, o_hbm):
        def body(x_vmem, i_vmem):
            pltpu.sync_copy(x_vmem, o_hbm.at[i_vmem.at[0]])

        pltpu.emit_pipeline(
            body,
            grid=(NPAD // 128,),
            in_specs=[
                pl.BlockSpec((128, CP), lambda i: (i, 0)),
                pl.BlockSpec((1, 128), lambda i: (0, i)),
            ],
            out_specs=[],
            core_axis_name="s",
            dimension_semantics=(pltpu.PARALLEL,),
        )(x_hbm, i_hbm)

    return k(tokens_pad, dest)


def _sc_gather(y_sorted, dest):
    mesh = plsc.VectorSubcoreMesh(core_axis_name="c", subcore_axis_name="s")

    @functools.partial(
        pl.kernel,
        out_type=jax.ShapeDtypeStruct((NPAD, CP), jnp.float32),
        mesh=mesh)
    def k(y_hbm, i_hbm, o_hbm):
        def body(i_vmem, o_vmem):
            pltpu.sync_copy(y_hbm.at[i_vmem.at[0]], o_vmem)

        pltpu.emit_pipeline(
            body,
            grid=(NPAD // 128,),
            in_specs=[pl.BlockSpec((1, 128), lambda i: (0, i))],
            out_specs=[pl.BlockSpec((128, CP), lambda i: (i, 0))],
            core_axis_name="s",
            dimension_semantics=(pltpu.PARALLEL,),
        )(i_hbm, o_hbm)

    return k(y_sorted, dest)


def _expert_body(co_ref, tok_ref, w1_ref, b1_ref, w2_ref, b2_ref,
                 scale_ref, out_ref):
    scale = scale_ref[0, 0]

    def mlp(t, e_w1, e_b1, e_w2, e_b2):
        h1 = jax.lax.dot_general(
            t.astype(jnp.bfloat16), e_w1, (((1,), (0,)), ((), ())),
            preferred_element_type=jnp.float32) + e_b1
        h1 = 0.5 * h1 * (1.0 + jax.lax.erf(h1 * 0.7071067811865476))
        ye = jax.lax.dot_general(
            h1.astype(jnp.bfloat16), e_w2, (((1,), (0,)), ((), ())),
            preferred_element_type=jnp.float32) + e_b2
        return t + scale * ye

    for e in range(E):
        t = tok_ref[pl.ds(e * CAP, CAP), :C]
        out_ref[pl.ds(e * CAP, CAP), :C] = mlp(
            t, w1_ref[e], b1_ref[e], w2_ref[e], b2_ref[e])

    novf = co_ref[NCHUNK_OVF]

    def step(j, carry):
        e = co_ref[j]
        t = tok_ref[pl.ds(OVF_BASE + j * RB, RB), :C]
        out_ref[pl.ds(OVF_BASE + j * RB, RB), :C] = mlp(
            t, w1_ref[e], b1_ref[e], w2_ref[e], b2_ref[e])
        return carry

    jax.lax.fori_loop(0, novf, step, 0)


def _expert_compute(co, sorted_tokens, W1, b1, W2, b2, scale):
    grid_spec = pltpu.PrefetchScalarGridSpec(
        num_scalar_prefetch=1,
        grid=(1,),
        in_specs=[
            pl.BlockSpec((NBUF, CP), lambda i, co: (0, 0)),
            pl.BlockSpec((E, C, HID), lambda i, co: (0, 0, 0)),
            pl.BlockSpec((E, 1, HID), lambda i, co: (0, 0, 0)),
            pl.BlockSpec((E, HID, C), lambda i, co: (0, 0, 0)),
            pl.BlockSpec((E, 1, C), lambda i, co: (0, 0, 0)),
            pl.BlockSpec((1, 1), lambda i, co: (0, 0)),
        ],
        out_specs=pl.BlockSpec((NBUF, CP), lambda i, co: (0, 0)),
    )
    return pl.pallas_call(
        _expert_body,
        grid_spec=grid_spec,
        out_shape=jax.ShapeDtypeStruct((NBUF, CP), jnp.float32),
    )(co, sorted_tokens,
      jnp.transpose(W1.astype(jnp.bfloat16), (0, 2, 1)),
      b1.reshape(E, 1, HID),
      jnp.transpose(W2.astype(jnp.bfloat16), (0, 2, 1)),
      b2.reshape(E, 1, C), scale.reshape(1, 1))


def kernel(x, Wr, br, W1, b1, W2, b2, scale):
    b, c, h, w = x.shape
    tokens = jnp.transpose(x, (0, 2, 3, 1)).reshape(b * h * w, c)
    tokens_pad = jnp.zeros((NPAD, CP), jnp.float32).at[:N, :C].set(tokens)

    dest, co = _router_meta(tokens_pad, Wr, br)
    sorted_tokens = _sc_scatter(tokens_pad, dest)
    y_sorted = _expert_compute(co.reshape(NCHUNK_OVF + 1), sorted_tokens,
                               W1, b1, W2, b2, scale)
    y_tokens = _sc_gather(y_sorted, dest)

    return jnp.transpose(y_tokens[:N, :C].reshape(b, h, w, c), (0, 3, 1, 2))


# speedup vs baseline: 1.0728x; 1.0605x over previous
import functools

import jax
import jax.numpy as jnp
from jax.experimental import pallas as pl
from jax.experimental.pallas import tpu as pltpu
from jax.experimental.pallas import tpu_sc as plsc

N = 784
NPAD = 896
C = 96
CP = 128
E = 64
HID = 192
CAP = 32
RB = 8
PRIM = E * CAP
OVFSLOTS = 1216
NCHUNK_OVF = OVFSLOTS // RB
OVF_BASE = PRIM
TRASH = PRIM + OVFSLOTS
NBUF = TRASH + (NPAD - N)


def _meta_body(tok_ref, wr_ref, br_ref, dest_ref, co_ref):
    logits = jax.lax.dot_general(
        wr_ref[:], tok_ref[:, :C], (((1,), (1,)), ((), ())),
        preferred_element_type=jnp.float32) + br_ref[:]
    maxv = jnp.max(logits, axis=0, keepdims=True)
    sub = jax.lax.broadcasted_iota(jnp.int32, (E, NPAD), 0)
    eidx = jnp.min(jnp.where(logits >= maxv, sub, E), axis=0,
                   keepdims=True)
    lane = jax.lax.broadcasted_iota(jnp.int32, (E, NPAD), 1)
    onehot = ((sub == eidx) & (lane < N)).astype(jnp.float32)

    rp = jax.lax.broadcasted_iota(jnp.int32, (NPAD, NPAD), 0)
    rq = jax.lax.broadcasted_iota(jnp.int32, (NPAD, NPAD), 1)
    ut = (rp < rq).astype(jnp.float32)
    cum = jax.lax.dot_general(
        onehot, ut, (((1,), (0,)), ((), ())),
        preferred_element_type=jnp.float32)
    rank_row = jnp.sum(onehot * cum, axis=0, keepdims=True)

    counts = jnp.sum(onehot, axis=1, keepdims=True)
    ovf = jnp.maximum(counts - float(CAP), 0.0)
    pco = jnp.floor((ovf + 7.0) * 0.125) * 8.0
    r64 = jax.lax.broadcasted_iota(jnp.int32, (E, E), 0)
    c64 = jax.lax.broadcasted_iota(jnp.int32, (E, E), 1)
    lt = (c64 < r64).astype(jnp.float32)
    offo = jax.lax.dot_general(
        lt, pco, (((1,), (0,)), ((), ())),
        preferred_element_type=jnp.float32)
    offo_row = jnp.sum(onehot * offo, axis=0, keepdims=True)
    eidx_f = jnp.sum(onehot * sub.astype(jnp.float32), axis=0,
                     keepdims=True)

    prim = eidx_f * float(CAP) + rank_row
    ovfd = float(OVF_BASE) + offo_row + rank_row - float(CAP)
    dest = jnp.where(rank_row < float(CAP), prim, ovfd).astype(jnp.int32)
    lane1 = jax.lax.broadcasted_iota(jnp.int32, (1, NPAD), 1)
    dest_ref[:] = jnp.where(lane1 < N, dest, TRASH + lane1 - N)

    offo_end = (offo + pco).astype(jnp.int32)
    cj = jax.lax.broadcasted_iota(jnp.int32, (E, NCHUNK_OVF), 1) * RB
    ce = jnp.sum((offo_end <= cj).astype(jnp.int32), axis=0, keepdims=True)
    ce = jnp.minimum(ce, E - 1)
    novf = jnp.sum(pco, axis=0, keepdims=True).astype(jnp.int32) // RB
    lanec = jax.lax.broadcasted_iota(jnp.int32, (1, NCHUNK_OVF + 1), 1)
    co_ref[:] = jnp.where(lanec < NCHUNK_OVF,
                          jnp.pad(ce, ((0, 0), (0, 1))),
                          jnp.broadcast_to(novf, (1, NCHUNK_OVF + 1)))


def _router_meta(tokens_pad, Wr, br):
    return pl.pallas_call(
        _meta_body,
        in_specs=[
            pl.BlockSpec((NPAD, CP), lambda: (0, 0)),
            pl.BlockSpec((E, C), lambda: (0, 0)),
            pl.BlockSpec((E, 1), lambda: (0, 0)),
        ],
        out_specs=[
            pl.BlockSpec((1, NPAD), lambda: (0, 0)),
            pl.BlockSpec((1, NCHUNK_OVF + 1), lambda: (0, 0)),
        ],
        out_shape=[
            jax.ShapeDtypeStruct((1, NPAD), jnp.int32),
            jax.ShapeDtypeStruct((1, NCHUNK_OVF + 1), jnp.int32),
        ],
    )(tokens_pad, Wr, br.reshape(E, 1))


def _sc_scatter(tokens_pad, dest):
    mesh = plsc.VectorSubcoreMesh(core_axis_name="c", subcore_axis_name="s")

    @functools.partial(
        pl.kernel,
        out_type=jax.ShapeDtypeStruct((NBUF, CP), jnp.float32),
        mesh=mesh)
    def k(x_hbm, i_hbm, o_hbm):
        def body(x_vmem, i_vmem):
            pltpu.sync_copy(x_vmem, o_hbm.at[i_vmem.at[0]])

        pltpu.emit_pipeline(
            body,
            grid=(NPAD // 128,),
            in_specs=[
                pl.BlockSpec((128, CP), lambda i: (i, 0)),
                pl.BlockSpec((1, 128), lambda i: (0, i)),
            ],
            out_specs=[],
            core_axis_name="s",
            dimension_semantics=(pltpu.PARALLEL,),
        )(x_hbm, i_hbm)

    return k(tokens_pad, dest)


def _sc_gather(y_sorted, dest):
    mesh = plsc.VectorSubcoreMesh(core_axis_name="c", subcore_axis_name="s")

    @functools.partial(
        pl.kernel,
        out_type=jax.ShapeDtypeStruct((NPAD, CP), jnp.float32),
        mesh=mesh)
    def k(y_hbm, i_hbm, o_hbm):
        def body(i_vmem, o_vmem):
            pltpu.sync_copy(y_hbm.at[i_vmem.at[0]], o_vmem)

        pltpu.emit_pipeline(
            body,
            grid=(NPAD // 128,),
            in_specs=[pl.BlockSpec((1, 128), lambda i: (0, i))],
            out_specs=[pl.BlockSpec((128, CP), lambda i: (i, 0))],
            core_axis_name="s",
            dimension_semantics=(pltpu.PARALLEL,),
        )(i_hbm, o_hbm)

    return k(y_sorted, dest)


def _expert_body(co_ref, tok_ref, w1_ref, b1_ref, w2_ref, b2_ref,
                 scale_ref, out_ref, h1_ref):
    scale = scale_ref[0, 0]

    for e in range(E):
        t = tok_ref[pl.ds(e * CAP, CAP), :C]
        h1_ref[pl.ds(e * CAP, CAP), :] = jax.lax.dot_general(
            t.astype(jnp.bfloat16), w1_ref[e], (((1,), (0,)), ((), ())),
            preferred_element_type=jnp.float32) + b1_ref[e]

    h1 = h1_ref[:]
    h1_ref[:] = (0.5 * h1 * (1.0 + jax.lax.erf(h1 * 0.7071067811865476))
                 ).astype(jnp.bfloat16).astype(jnp.float32)

    for e in range(E):
        t = tok_ref[pl.ds(e * CAP, CAP), :C]
        ye = jax.lax.dot_general(
            h1_ref[pl.ds(e * CAP, CAP), :].astype(jnp.bfloat16),
            w2_ref[e], (((1,), (0,)), ((), ())),
            preferred_element_type=jnp.float32) + b2_ref[e]
        out_ref[pl.ds(e * CAP, CAP), :C] = t + scale * ye

    def mlp(t, e_w1, e_b1, e_w2, e_b2):
        h1 = jax.lax.dot_general(
            t.astype(jnp.bfloat16), e_w1, (((1,), (0,)), ((), ())),
            preferred_element_type=jnp.float32) + e_b1
        h1 = 0.5 * h1 * (1.0 + jax.lax.erf(h1 * 0.7071067811865476))
        ye = jax.lax.dot_general(
            h1.astype(jnp.bfloat16), e_w2, (((1,), (0,)), ((), ())),
            preferred_element_type=jnp.float32) + e_b2
        return t + scale * ye

    novf = co_ref[NCHUNK_OVF]

    def step(j, carry):
        e = co_ref[j]
        t = tok_ref[pl.ds(OVF_BASE + j * RB, RB), :C]
        out_ref[pl.ds(OVF_BASE + j * RB, RB), :C] = mlp(
            t, w1_ref[e], b1_ref[e], w2_ref[e], b2_ref[e])
        return carry

    jax.lax.fori_loop(0, novf, step, 0)


def _expert_compute(co, sorted_tokens, W1, b1, W2, b2, scale):
    grid_spec = pltpu.PrefetchScalarGridSpec(
        num_scalar_prefetch=1,
        grid=(1,),
        in_specs=[
            pl.BlockSpec((NBUF, CP), lambda i, co: (0, 0)),
            pl.BlockSpec((E, C, HID), lambda i, co: (0, 0, 0)),
            pl.BlockSpec((E, 1, HID), lambda i, co: (0, 0, 0)),
            pl.BlockSpec((E, HID, C), lambda i, co: (0, 0, 0)),
            pl.BlockSpec((E, 1, C), lambda i, co: (0, 0, 0)),
            pl.BlockSpec((1, 1), lambda i, co: (0, 0)),
        ],
        out_specs=pl.BlockSpec((NBUF, CP), lambda i, co: (0, 0)),
        scratch_shapes=[pltpu.VMEM((PRIM, HID), jnp.float32)],
    )
    return pl.pallas_call(
        _expert_body,
        grid_spec=grid_spec,
        out_shape=jax.ShapeDtypeStruct((NBUF, CP), jnp.float32),
    )(co, sorted_tokens,
      jnp.transpose(W1.astype(jnp.bfloat16), (0, 2, 1)),
      b1.reshape(E, 1, HID),
      jnp.transpose(W2.astype(jnp.bfloat16), (0, 2, 1)),
      b2.reshape(E, 1, C), scale.reshape(1, 1))


def kernel(x, Wr, br, W1, b1, W2, b2, scale):
    b, c, h, w = x.shape
    tokens = jnp.transpose(x, (0, 2, 3, 1)).reshape(b * h * w, c)
    tokens_pad = jnp.zeros((NPAD, CP), jnp.float32).at[:N, :C].set(tokens)

    dest, co = _router_meta(tokens_pad, Wr, br)
    sorted_tokens = _sc_scatter(tokens_pad, dest)
    y_sorted = _expert_compute(co.reshape(NCHUNK_OVF + 1), sorted_tokens,
                               W1, b1, W2, b2, scale)
    y_tokens = _sc_gather(y_sorted, dest)

    return jnp.transpose(y_tokens[:N, :C].reshape(b, h, w, c), (0, 3, 1, 2))
